# baseline (speedup 1.0000x reference)
import functools

import jax
import jax.numpy as jnp
from jax import lax
from jax.experimental import pallas as pl
from jax.experimental.pallas import tpu as pltpu
from jax.experimental.pallas import tpu_sc as plsc

_N = 6
_B_SC = 32


def _partsum_kernel(x_ref, o_ref):
    BB, L, C = x_ref.shape
    ratio = L // _N
    for bb in range(BB):
        xb = x_ref[bb]
        parts = [
            jnp.sum(xb[i * ratio:(i + 1) * ratio, :], axis=0, keepdims=True)
            for i in range(_N)
        ]
        o_ref[bb] = jnp.concatenate(parts, axis=0)


def _sc_pieces(ratio):
    pieces = []
    h = ((ratio // 2) // 8) * 8 + 8
    for p in range(_N):
        for (s, l, first) in ((p * ratio, h, True),
                              (p * ratio + h, ratio - h, False)):
            astart = (s // 8) * 8
            off = s - astart
            alen = ((off + l + 7) // 8) * 8
            pieces.append((p, astart, off, alen, l, first))
    return pieces


def _sc_partsum(b_base, ratio, x_hbm, out_hbm, buf0, buf1, acc, sem0, sem1):
    C = acc.shape[1]
    nv = C // 16
    wid = lax.axis_index("s") * 2 + lax.axis_index("c")
    b = b_base + wid
    bufs = (buf0, buf1)
    sems = (sem0, sem1)
    pieces = _sc_pieces(ratio)

    def start_copy(i):
        _, astart, _, alen, _, _ = pieces[i]
        return pltpu.async_copy(
            x_hbm.at[b, pl.ds(astart, alen), :],
            bufs[i % 2].at[pl.ds(0, alen), :], sems[i % 2])

    copies = [start_copy(0)]
    accs = None
    for i, (p, astart, off, alen, l, first) in enumerate(pieces):
        if i + 1 < len(pieces):
            copies.append(start_copy(i + 1))
        copies[i].wait()
        buf = bufs[i % 2]
        if first:
            accs = tuple(jnp.zeros((16,), jnp.float32) for _ in range(nv))

        def body(k, a, buf=buf, off=off):
            r = off + k * 2
            a = tuple(a[v] + buf[r, pl.ds(v * 16, 16)] for v in range(nv))
            return tuple(a[v] + buf[r + 1, pl.ds(v * 16, 16)]
                         for v in range(nv))

        accs = lax.fori_loop(0, l // 2, body, accs)
        if l % 2:
            r = off + (l // 2) * 2
            accs = tuple(accs[v] + buf[r, pl.ds(v * 16, 16)]
                         for v in range(nv))
        if not first:
            for v in range(nv):
                acc[p, pl.ds(v * 16, 16)] = accs[v]
    pltpu.sync_copy(acc, out_hbm.at[wid])


def _finish_kernel(ps_ref, w_ref, b_ref, o_ref, *, ratio):
    B, N, C = ps_ref.shape
    pf = ps_ref[...] * (1.0 / ratio)
    p = [pf[:, n, :] for n in range(N)]

    cols = [[None] * N for _ in range(N)]
    zero = jnp.zeros((B, 1), jnp.float32)
    for n in range(N):
        cols[n][n] = zero
        for m in range(n + 1, N):
            d = p[n] - p[m]
            s = jnp.sum(d * d, axis=-1, keepdims=True)
            cols[n][m] = s
            cols[m][n] = s
    D = jnp.concatenate(
        [cols[n][m] for n in range(N) for m in range(N)], axis=1)
    Dt = D.T
    row = [Dt[i:i + 1, :] for i in range(N * N)]

    deg = []
    for m in range(N):
        dm = jnp.zeros((1, B), jnp.float32)
        for n in range(N):
            r = jnp.zeros((1, B), jnp.float32)
            for j in range(N):
                if j == m:
                    continue
                if j < m:
                    r += (row[n * N + j] <= row[n * N + m]).astype(jnp.float32)
                else:
                    r += (row[n * N + j] < row[n * N + m]).astype(jnp.float32)
            dm += (r <= 2.5).astype(jnp.float32)
        deg.append(dm)
    Cmat = jnp.concatenate(deg, axis=0)
    c = Cmat.T * (1.0 / ((3.0 + 1e-6) * N))

    g = c[:, 0:1] * p[0]
    for m in range(1, N):
        g = g + c[:, m:m + 1] * p[m]
    mean_pf = jnp.sum(pf, axis=1) * (1.0 / N)
    out = jax.lax.dot_general(
        g, w_ref[...], (((1,), (1,)), ((), ())),
        preferred_element_type=jnp.float32)
    o_ref[...] = out + b_ref[...] + mean_pf


@jax.jit
def kernel(x, W, b):
    B, L, C = x.shape
    ratio = L // _N
    B_TC = B - _B_SC
    BB = 4
    ps_tc = pl.pallas_call(
        _partsum_kernel,
        grid=(B_TC // BB,),
        in_specs=[pl.BlockSpec((BB, L, C), lambda i: (i, 0, 0))],
        out_specs=pl.BlockSpec((BB, _N, C), lambda i: (i, 0, 0)),
        out_shape=jax.ShapeDtypeStruct((B_TC, _N, C), x.dtype),
        compiler_params=pltpu.CompilerParams(
            dimension_semantics=("arbitrary",),
        ),
    )(x)

    mesh = plsc.VectorSubcoreMesh(core_axis_name="c", subcore_axis_name="s")
    sc_call = functools.partial(
        pl.kernel,
        mesh=mesh,
        out_type=jax.ShapeDtypeStruct((_B_SC, _N, C), jnp.float32),
        scratch_types=[
            pltpu.VMEM((192, C), jnp.float32),
            pltpu.VMEM((192, C), jnp.float32),
            pltpu.VMEM((_N, C), jnp.float32),
            pltpu.SemaphoreType.DMA,
            pltpu.SemaphoreType.DMA,
        ],
    )(functools.partial(_sc_partsum, B_TC, ratio))
    ps_sc = sc_call(x)

    ps = jnp.concatenate([ps_tc, ps_sc], axis=0)

    out = pl.pallas_call(
        functools.partial(_finish_kernel, ratio=ratio),
        in_specs=[
            pl.BlockSpec((B, _N, C), lambda: (0, 0, 0)),
            pl.BlockSpec((C, C), lambda: (0, 0)),
            pl.BlockSpec((1, C), lambda: (0, 0)),
        ],
        out_specs=pl.BlockSpec((B, C), lambda: (0, 0)),
        out_shape=jax.ShapeDtypeStruct((B, C), x.dtype),
    )(ps, W, b.reshape(1, C))
    return out


# speedup vs baseline: 1.3029x; 1.3029x over previous
import functools

import jax
import jax.numpy as jnp
from jax.experimental import pallas as pl
from jax.experimental.pallas import tpu as pltpu

_N = 6


def _tgam_kernel(x_ref, w_ref, b_ref, o_ref, ps_ref):
    BB, L, C = x_ref.shape
    B = ps_ref.shape[1]
    ratio = L // _N
    i = pl.program_id(0)
    nsteps = pl.num_programs(0)

    for n in range(_N):
        rows = [
            jnp.sum(x_ref[bb, n * ratio:(n + 1) * ratio, :], axis=0,
                    keepdims=True)
            for bb in range(BB)
        ]
        ps_ref[n, pl.ds(pl.multiple_of(i * BB, BB), BB), :] = (
            jnp.concatenate(rows, axis=0))

    @pl.when(i == nsteps - 1)
    def _finish():
        inv = 1.0 / ratio
        p = [ps_ref[n] * inv for n in range(_N)]

        cols = [[None] * _N for _ in range(_N)]
        zero = jnp.zeros((B, 1), jnp.float32)
        for n in range(_N):
            cols[n][n] = zero
            for m in range(n + 1, _N):
                d = p[n] - p[m]
                s = jnp.sum(d * d, axis=-1, keepdims=True)
                cols[n][m] = s
                cols[m][n] = s
        D = jnp.concatenate(
            [cols[n][m] for n in range(_N) for m in range(_N)], axis=1)
        Dt = D.T
        row = [Dt[k:k + 1, :] for k in range(_N * _N)]

        deg = []
        for m in range(_N):
            dm = jnp.zeros((1, B), jnp.float32)
            for n in range(_N):
                r = jnp.zeros((1, B), jnp.float32)
                for j in range(_N):
                    if j == m:
                        continue
                    if j < m:
                        r += (row[n * _N + j] <= row[n * _N + m]).astype(
                            jnp.float32)
                    else:
                        r += (row[n * _N + j] < row[n * _N + m]).astype(
                            jnp.float32)
                dm += (r <= 2.5).astype(jnp.float32)
            deg.append(dm)
        Cmat = jnp.concatenate(deg, axis=0)
        c = Cmat.T * (1.0 / ((3.0 + 1e-6) * _N))

        g = c[:, 0:1] * p[0]
        for m in range(1, _N):
            g = g + c[:, m:m + 1] * p[m]
        mean_pf = p[0]
        for m in range(1, _N):
            mean_pf = mean_pf + p[m]
        mean_pf = mean_pf * (1.0 / _N)
        out = jax.lax.dot_general(
            g, w_ref[...], (((1,), (1,)), ((), ())),
            preferred_element_type=jnp.float32)
        o_ref[...] = out + b_ref[...] + mean_pf


@jax.jit
def kernel(x, W, b):
    B, L, C = x.shape
    BB = 8
    out = pl.pallas_call(
        _tgam_kernel,
        grid=(B // BB,),
        in_specs=[
            pl.BlockSpec((BB, L, C), lambda i: (i, 0, 0)),
            pl.BlockSpec((C, C), lambda i: (0, 0)),
            pl.BlockSpec((1, C), lambda i: (0, 0)),
        ],
        out_specs=pl.BlockSpec((B, C), lambda i: (0, 0)),
        out_shape=jax.ShapeDtypeStruct((B, C), x.dtype),
        scratch_shapes=[pltpu.VMEM((_N, B, C), jnp.float32)],
        compiler_params=pltpu.CompilerParams(
            dimension_semantics=("arbitrary",),
        ),
    )(x, W, b.reshape(1, C))
    return out


# speedup vs baseline: 1.3191x; 1.0125x over previous
import functools

import jax
import jax.numpy as jnp
from jax.experimental import pallas as pl
from jax.experimental.pallas import tpu as pltpu

_N = 6


def _tgam_kernel(x_ref, w_ref, b_ref, o_ref, ps_ref):
    BB, L, C = x_ref.shape
    B = ps_ref.shape[1] * ps_ref.shape[2]
    ratio = L // _N
    i = pl.program_id(0)
    nsteps = pl.num_programs(0)

    for n in range(_N):
        rows = [
            jnp.sum(x_ref[bb, n * ratio:(n + 1) * ratio, :], axis=0,
                    keepdims=True)
            for bb in range(BB)
        ]
        ps_ref[n, i] = jnp.concatenate(rows, axis=0)

    @pl.when(i == nsteps - 1)
    def _finish():
        inv = 1.0 / ratio
        p = [ps_ref[n].reshape(B, C) * inv for n in range(_N)]

        cols = [[None] * _N for _ in range(_N)]
        zero = jnp.zeros((B, 1), jnp.float32)
        for n in range(_N):
            cols[n][n] = zero
            for m in range(n + 1, _N):
                d = p[n] - p[m]
                s = jnp.sum(d * d, axis=-1, keepdims=True)
                cols[n][m] = s
                cols[m][n] = s
        D = jnp.concatenate(
            [cols[n][m] for n in range(_N) for m in range(_N)], axis=1)
        Dt = D.T
        row = [Dt[k:k + 1, :] for k in range(_N * _N)]

        deg = []
        for m in range(_N):
            dm = jnp.zeros((1, B), jnp.float32)
            for n in range(_N):
                r = jnp.zeros((1, B), jnp.float32)
                for j in range(_N):
                    if j == m:
                        continue
                    if j < m:
                        r += (row[n * _N + j] <= row[n * _N + m]).astype(
                            jnp.float32)
                    else:
                        r += (row[n * _N + j] < row[n * _N + m]).astype(
                            jnp.float32)
                dm += (r <= 2.5).astype(jnp.float32)
            deg.append(dm)
        Cmat = jnp.concatenate(deg, axis=0)
        c = Cmat.T * (1.0 / ((3.0 + 1e-6) * _N))

        g = c[:, 0:1] * p[0]
        for m in range(1, _N):
            g = g + c[:, m:m + 1] * p[m]
        mean_pf = p[0]
        for m in range(1, _N):
            mean_pf = mean_pf + p[m]
        mean_pf = mean_pf * (1.0 / _N)
        out = jax.lax.dot_general(
            g, w_ref[...], (((1,), (1,)), ((), ())),
            preferred_element_type=jnp.float32)
        o_ref[...] = out + b_ref[...] + mean_pf


@jax.jit
def kernel(x, W, b):
    B, L, C = x.shape
    BB = 4
    out = pl.pallas_call(
        _tgam_kernel,
        grid=(B // BB,),
        in_specs=[
            pl.BlockSpec((BB, L, C), lambda i: (i, 0, 0)),
            pl.BlockSpec((C, C), lambda i: (0, 0)),
            pl.BlockSpec((1, C), lambda i: (0, 0)),
        ],
        out_specs=pl.BlockSpec((B, C), lambda i: (0, 0)),
        out_shape=jax.ShapeDtypeStruct((B, C), x.dtype),
        scratch_shapes=[pltpu.VMEM((_N, B // BB, BB, C), jnp.float32)],
        compiler_params=pltpu.CompilerParams(
            dimension_semantics=("arbitrary",),
        ),
    )(x, W, b.reshape(1, C))
    return out
